# baseline (speedup 1.0000x reference)
import functools

import jax
import jax.numpy as jnp
from jax import lax
from jax.experimental import pallas as pl
from jax.experimental.pallas import tpu as pltpu
from jax.experimental.pallas import tpu_sc as plsc

_WTC = 4
_NWIN = 62
_CAP = 1024
_PCAP = 512
_SBC = 160
_WCAP = 48
_DUMP = 1024


@functools.lru_cache(maxsize=None)
def _make_kernels(num_rows: int, embed_dim: int, batch: int):
    info = plsc.get_sparse_core_info()
    nc, ns = info.num_cores, info.num_subcores
    nw = nc * ns
    tcols = (num_rows + 127) // 128
    tq, tr = divmod(tcols, nw)
    max_base = tcols - _WTC
    scat_rows = batch + _DUMP
    n_vecs = batch // 16
    wcols = _WTC * 128
    mesh = plsc.VectorSubcoreMesh(core_axis_name="c", subcore_axis_name="s")

    @functools.partial(
        pl.kernel,
        mesh=mesh,
        compiler_params=pltpu.CompilerParams(needs_layout_passes=False),
        out_type=jax.ShapeDtypeStruct((scat_rows, 128), jnp.float32),
        scratch_types=[
            pltpu.VMEM((batch // 128, 128), jnp.int32),
            pltpu.VMEM((2 * embed_dim, wcols), jnp.float32),
            pltpu.VMEM((_PCAP, 128), jnp.float32),
            pltpu.VMEM((_CAP // 128, 128), jnp.int32),
            pltpu.VMEM((_CAP // 128, 128), jnp.int32),
            pltpu.VMEM((8 * _SBC // 128 + 1, 128), jnp.int32),
            pltpu.VMEM((8 * _SBC // 128 + 1, 128), jnp.int32),
            pltpu.VMEM((1, 128), jnp.int32),
            pltpu.VMEM((1, 128), jnp.int32),
            pltpu.VMEM((_PCAP,), jnp.int32),
            pltpu.VMEM((1, 128), jnp.int32),
            pltpu.SemaphoreType.DMA,
            pltpu.SemaphoreType.DMA,
        ],
    )
    def scan_kernel(idx_hbm, table_hbm, scat_hbm, idx_v, winbuf, rows_v,
                    hit_r, hit_b, sb_r, sb_b, w_r, w_b, sc_b, cnts_v,
                    sem0, sem1):
        wid = lax.axis_index("s") * nc + lax.axis_index("c")
        lo_tc = wid * tq + jnp.minimum(wid, tr)
        lo_u = lo_tc * 128
        hi_u = lo_u + (tq + jnp.where(wid < tr, 1, 0)) * 128
        lanes = lax.iota(jnp.int32, 16)
        sems = (sem0, sem1)

        pltpu.sync_copy(idx_hbm, idx_v)

        def fire(w, parity):
            base = jnp.minimum(lo_tc + _WTC * w, max_base)
            off = pl.multiple_of(base * 128, 128)
            pltpu.async_copy(
                table_hbm.at[:, pl.ds(off, wcols)],
                winbuf.at[pl.ds(parity * embed_dim, embed_dim)],
                sems[parity],
            )

        def wait(parity):
            pltpu.make_async_copy(
                table_hbm.at[:, pl.ds(0, wcols)],
                winbuf.at[pl.ds(0, embed_dim)],
                sems[parity],
            ).wait()

        fire(0, 0)
        fire(1, 1)

        def scan_body(q, nh):
            iv = idx_v[lax.shift_right_logical(q, 3), pl.ds((q & 7) * 16, 16)]
            bv = q * 16 + lanes
            m = (iv >= lo_u) & (iv < hi_u)

            def hit_case(nh):
                cs = plsc.cumsum(jnp.where(m, 1, 0))
                pos = jnp.minimum(nh + cs - 1, _CAP - 1)
                pr, pc = lax.shift_right_logical(pos, 7), pos & 127
                plsc.store_scatter(hit_r, [pr, pc], iv, mask=m)
                plsc.store_scatter(hit_b, [pr, pc], bv, mask=m)
                return nh + cs[15]

            return lax.cond(jnp.any(m), hit_case, lambda n: n, nh)

        nh = lax.fori_loop(0, n_vecs, scan_body, 0)

        def fill_dump(k, carry):
            slot = k * 16 + lanes
            plsc.store_scatter(sc_b, [slot], batch + (slot & (_DUMP - 1)))
            return carry

        lax.fori_loop(0, _PCAP // 16, fill_dump, 0)

        def sb_body(k, carry):
            valid = (k * 16 + lanes) < nh
            o = k * 16
            orow, ocol = lax.shift_right_logical(o, 7), o & 127
            rv = hit_r[orow, pl.ds(ocol, 16)]
            bv = hit_b[orow, pl.ds(ocol, 16)]
            wv = lax.shift_right_logical(
                lax.shift_right_logical(rv, 7) - lo_tc, 2)
            sb = lax.shift_right_logical(wv, 3)
            new = []
            for s in range(8):
                c_s = carry[s]
                m = (sb == s) & valid
                cs = plsc.cumsum(jnp.where(m, 1, 0))
                pos = jnp.minimum(s * _SBC + c_s + cs - 1,
                                  s * _SBC + _SBC - 1)
                pr, pc = lax.shift_right_logical(pos, 7), pos & 127
                plsc.store_scatter(sb_r, [pr, pc], rv, mask=m)
                plsc.store_scatter(sb_b, [pr, pc], bv, mask=m)
                new.append(c_s + cs[15])
            return tuple(new)

        cnts = lax.fori_loop(0, lax.shift_right_logical(nh + 15, 4),
                             sb_body, (0,) * 8)
        for s in range(8):
            plsc.store_scatter(cnts_v, [0 * lanes, s + 0 * lanes],
                               cnts[s] + 0 * lanes, mask=lanes < 1)

        def process(w, parity, ecnt):
            s = lax.shift_right_logical(w, 3)
            cnt_s = plsc.load_gather(cnts_v, [0 * lanes, s + 0 * lanes])[0]

            def filt_body(k, wc):
                valid = (k * 16 + lanes) < cnt_s
                o = s * _SBC + k * 16
                orow, ocol = lax.shift_right_logical(o, 7), o & 127
                rv = sb_r[orow, pl.ds(ocol, 16)]
                bv = sb_b[orow, pl.ds(ocol, 16)]
                m = (lax.shift_right_logical(
                    lax.shift_right_logical(rv, 7) - lo_tc, 2) == w) & valid
                cs = plsc.cumsum(jnp.where(m, 1, 0))
                pos = jnp.minimum(wc + cs - 1, _WCAP - 1)
                pr, pc = pos * 0, pos
                plsc.store_scatter(w_r, [pr, pc], rv, mask=m)
                plsc.store_scatter(w_b, [pr, pc], bv, mask=m)
                return wc + cs[15]

            wcnt = lax.fori_loop(
                0, lax.shift_right_logical(cnt_s + 15, 4), filt_body, 0)

            wait(parity)
            base_u = jnp.minimum(lo_tc + _WTC * w, max_base) * 128

            def ex_body(k, ec):
                valid = (k * 16 + lanes) < wcnt
                rv = w_r[0, pl.ds(k * 16, 16)]
                bv = w_b[0, pl.ds(k * 16, 16)]
                rloc_v = (rv - base_u) & (wcols - 1)
                cs = plsc.cumsum(jnp.where(valid, 1, 0))
                ecv = jnp.minimum(ec + cs - 1, _PCAP - 1)
                plsc.store_scatter(sc_b, [ecv], bv, mask=valid)
                for c in range(embed_dim):
                    vals = plsc.load_gather(
                        winbuf,
                        [parity * embed_dim + c + 0 * lanes, rloc_v],
                    )
                    plsc.store_scatter(
                        rows_v, [ecv, c + 0 * lanes], vals, mask=valid
                    )
                return ec + cs[15]

            return lax.fori_loop(
                0, lax.shift_right_logical(wcnt + 15, 4), ex_body, ecnt)

        def pair_a(hh, ecnt):
            w = 2 * hh
            ecnt = process(w, 0, ecnt)
            fire(w + 2, 0)
            ecnt = process(w + 1, 1, ecnt)
            fire(w + 3, 1)
            return ecnt

        ecnt = lax.fori_loop(0, 15, pair_a, 0)
        ecnt = process(30, 0, ecnt)
        fire(32, 0)
        ecnt = process(31, 1, ecnt)
        fire(33, 1)
        pltpu.sync_copy(rows_v, scat_hbm.at[sc_b])

        lax.fori_loop(0, _PCAP // 16, fill_dump, 0)

        def pair_b(hh, ecnt):
            w = 32 + 2 * hh
            ecnt = process(w, 0, ecnt)
            fire(w + 2, 0)
            ecnt = process(w + 1, 1, ecnt)
            fire(w + 3, 1)
            return ecnt

        ecnt = lax.fori_loop(0, 14, pair_b, 0)
        ecnt = process(60, 0, ecnt)
        ecnt = process(61, 1, ecnt)
        pltpu.sync_copy(rows_v, scat_hbm.at[sc_b])

    return scan_kernel


def kernel(inputs, user_table):
    batch, = inputs.shape
    num_rows, embed_dim = user_table.shape
    idx2d = inputs.astype(jnp.int32).reshape(batch // 128, 128)
    scan_k = _make_kernels(num_rows, embed_dim, batch)
    scat = scan_k(idx2d, user_table.T)
    return scat[:batch, :embed_dim]


# speedup vs baseline: 1.1300x; 1.1300x over previous
import functools

import jax
import jax.numpy as jnp
from jax import lax
from jax.experimental import pallas as pl
from jax.experimental.pallas import tpu as pltpu
from jax.experimental.pallas import tpu_sc as plsc

_WTC = 4
_NWIN = 62
_CAP = 1024
_PCAP = 512
_SBC = 160
_WCAP = 48
_DUMP = 1024


@functools.lru_cache(maxsize=None)
def _make_kernels(num_rows: int, embed_dim: int, batch: int):
    info = plsc.get_sparse_core_info()
    nc, ns = info.num_cores, info.num_subcores
    nw = nc * ns
    tcols = (num_rows + 127) // 128
    tq, tr = divmod(tcols, nw)
    max_base = tcols - _WTC
    scat_rows = batch + _DUMP
    n_vecs = batch // 16
    wcols = _WTC * 128
    mesh = plsc.VectorSubcoreMesh(core_axis_name="c", subcore_axis_name="s")

    @functools.partial(
        pl.kernel,
        mesh=mesh,
        compiler_params=pltpu.CompilerParams(needs_layout_passes=False),
        out_type=jax.ShapeDtypeStruct((scat_rows, 128), jnp.float32),
        scratch_types=[
            pltpu.VMEM((batch // 128, 128), jnp.int32),
            pltpu.VMEM((2 * embed_dim, wcols), jnp.float32),
            pltpu.VMEM((_PCAP, 128), jnp.float32),
            pltpu.VMEM((_CAP // 128, 128), jnp.int32),
            pltpu.VMEM((_CAP // 128, 128), jnp.int32),
            pltpu.VMEM((8 * _SBC // 128 + 1, 128), jnp.int32),
            pltpu.VMEM((8 * _SBC // 128 + 1, 128), jnp.int32),
            pltpu.VMEM((1, 128), jnp.int32),
            pltpu.VMEM((1, 128), jnp.int32),
            pltpu.VMEM((_PCAP,), jnp.int32),
            pltpu.VMEM((1, 128), jnp.int32),
            pltpu.SemaphoreType.DMA,
            pltpu.SemaphoreType.DMA,
        ],
    )
    def scan_kernel(idx_hbm, table_hbm, scat_hbm, idx_v, winbuf, rows_v,
                    hit_r, hit_b, sb_r, sb_b, w_r, w_b, sc_b, cnts_v,
                    sem0, sem1):
        wid = lax.axis_index("s") * nc + lax.axis_index("c")
        lo_tc = wid * tq + jnp.minimum(wid, tr)
        lo_u = lo_tc * 128
        hi_u = lo_u + (tq + jnp.where(wid < tr, 1, 0)) * 128
        lanes = lax.iota(jnp.int32, 16)
        sems = (sem0, sem1)

        pltpu.sync_copy(idx_hbm, idx_v)

        def fire(w, parity):
            base = jnp.minimum(lo_tc + _WTC * w, max_base)
            off = pl.multiple_of(base * 128, 128)
            pltpu.async_copy(
                table_hbm.at[:, pl.ds(off, wcols)],
                winbuf.at[pl.ds(parity * embed_dim, embed_dim)],
                sems[parity],
            )

        def wait(parity):
            pltpu.make_async_copy(
                table_hbm.at[:, pl.ds(0, wcols)],
                winbuf.at[pl.ds(0, embed_dim)],
                sems[parity],
            ).wait()

        fire(0, 0)
        fire(1, 1)

        def scan_body(qq, nh):
            row = lax.shift_right_logical(qq, 2)
            col = (qq & 3) * 32
            parts = []
            for u in range(2):
                iv = idx_v[row, pl.ds(col + u * 16, 16)]
                bv = (2 * qq + u) * 16 + lanes
                m = (iv >= lo_u) & (iv < hi_u)
                cs = plsc.cumsum(jnp.where(m, 1, 0))
                parts.append((iv, bv, m, cs))
            base = nh
            for iv, bv, m, cs in parts:
                pos = jnp.minimum(base + cs - 1, _CAP - 1)
                pr, pc = lax.shift_right_logical(pos, 7), pos & 127
                plsc.store_scatter(hit_r, [pr, pc], iv, mask=m)
                plsc.store_scatter(hit_b, [pr, pc], bv, mask=m)
                base = base + cs[15]
            return base

        nh = lax.fori_loop(0, n_vecs // 2, scan_body, 0)

        def fill_dump(k, carry):
            slot = k * 16 + lanes
            plsc.store_scatter(sc_b, [slot], batch + (slot & (_DUMP - 1)))
            return carry

        lax.fori_loop(0, _PCAP // 16, fill_dump, 0)

        def sb_body(k, carry):
            valid = (k * 16 + lanes) < nh
            o = k * 16
            orow, ocol = lax.shift_right_logical(o, 7), o & 127
            rv = hit_r[orow, pl.ds(ocol, 16)]
            bv = hit_b[orow, pl.ds(ocol, 16)]
            wv = lax.shift_right_logical(
                lax.shift_right_logical(rv, 7) - lo_tc, 2)
            sb = lax.shift_right_logical(wv, 3)
            new = []
            for s in range(8):
                c_s = carry[s]
                m = (sb == s) & valid
                cs = plsc.cumsum(jnp.where(m, 1, 0))
                pos = jnp.minimum(s * _SBC + c_s + cs - 1,
                                  s * _SBC + _SBC - 1)
                pr, pc = lax.shift_right_logical(pos, 7), pos & 127
                plsc.store_scatter(sb_r, [pr, pc], rv, mask=m)
                plsc.store_scatter(sb_b, [pr, pc], bv, mask=m)
                new.append(c_s + cs[15])
            return tuple(new)

        cnts = lax.fori_loop(0, lax.shift_right_logical(nh + 15, 4),
                             sb_body, (0,) * 8)
        for s in range(8):
            plsc.store_scatter(cnts_v, [0 * lanes, s + 0 * lanes],
                               cnts[s] + 0 * lanes, mask=lanes < 1)

        def process(w, parity, ecnt):
            s = lax.shift_right_logical(w, 3)
            cnt_s = plsc.load_gather(cnts_v, [0 * lanes, s + 0 * lanes])[0]

            def filt_body(k, wc):
                valid = (k * 16 + lanes) < cnt_s
                o = s * _SBC + k * 16
                orow, ocol = lax.shift_right_logical(o, 7), o & 127
                rv = sb_r[orow, pl.ds(ocol, 16)]
                bv = sb_b[orow, pl.ds(ocol, 16)]
                m = (lax.shift_right_logical(
                    lax.shift_right_logical(rv, 7) - lo_tc, 2) == w) & valid
                cs = plsc.cumsum(jnp.where(m, 1, 0))
                pos = jnp.minimum(wc + cs - 1, _WCAP - 1)
                pr, pc = pos * 0, pos
                plsc.store_scatter(w_r, [pr, pc], rv, mask=m)
                plsc.store_scatter(w_b, [pr, pc], bv, mask=m)
                return wc + cs[15]

            wcnt = lax.fori_loop(
                0, lax.shift_right_logical(cnt_s + 15, 4), filt_body, 0)

            wait(parity)
            base_u = jnp.minimum(lo_tc + _WTC * w, max_base) * 128

            def ex_body(k, ec):
                valid = (k * 16 + lanes) < wcnt
                rv = w_r[0, pl.ds(k * 16, 16)]
                bv = w_b[0, pl.ds(k * 16, 16)]
                rloc_v = (rv - base_u) & (wcols - 1)
                cs = plsc.cumsum(jnp.where(valid, 1, 0))
                ecv = jnp.minimum(ec + cs - 1, _PCAP - 1)
                plsc.store_scatter(sc_b, [ecv], bv, mask=valid)
                for c in range(embed_dim):
                    vals = plsc.load_gather(
                        winbuf,
                        [parity * embed_dim + c + 0 * lanes, rloc_v],
                    )
                    plsc.store_scatter(
                        rows_v, [ecv, c + 0 * lanes], vals, mask=valid
                    )
                return ec + cs[15]

            return lax.fori_loop(
                0, lax.shift_right_logical(wcnt + 15, 4), ex_body, ecnt)

        def pair_a(hh, ecnt):
            w = 2 * hh
            ecnt = process(w, 0, ecnt)
            fire(w + 2, 0)
            ecnt = process(w + 1, 1, ecnt)
            fire(w + 3, 1)
            return ecnt

        ecnt = lax.fori_loop(0, 15, pair_a, 0)
        ecnt = process(30, 0, ecnt)
        fire(32, 0)
        ecnt = process(31, 1, ecnt)
        fire(33, 1)
        pltpu.sync_copy(rows_v, scat_hbm.at[sc_b])

        lax.fori_loop(0, _PCAP // 16, fill_dump, 0)

        def pair_b(hh, ecnt):
            w = 32 + 2 * hh
            ecnt = process(w, 0, ecnt)
            fire(w + 2, 0)
            ecnt = process(w + 1, 1, ecnt)
            fire(w + 3, 1)
            return ecnt

        ecnt = lax.fori_loop(0, 14, pair_b, 0)
        ecnt = process(60, 0, ecnt)
        ecnt = process(61, 1, ecnt)
        pltpu.sync_copy(rows_v, scat_hbm.at[sc_b])

    return scan_kernel


def kernel(inputs, user_table):
    batch, = inputs.shape
    num_rows, embed_dim = user_table.shape
    idx2d = inputs.astype(jnp.int32).reshape(batch // 128, 128)
    scan_k = _make_kernels(num_rows, embed_dim, batch)
    scat = scan_k(idx2d, user_table.T)
    return scat[:batch, :embed_dim]
